# baseline (speedup 1.0000x reference)
import math

import jax
import jax.numpy as jnp
from jax.experimental import pallas as pl
from jax.experimental.pallas import tpu as pltpu

B = 2
T = 2048
C = 768
C3 = 3 * C
H = 12
HD = C // H
K_MAX = T // 2
F = 4 * C
GATE_H = 32

NEG = -1e30


_XP_BLK = 256


def _xproj_body(x0_ref, x1_ref, w_ref, b_ref, o_ref):
    w = w_ref[...]
    b = b_ref[...]
    r0 = jnp.dot(x0_ref[0], w, preferred_element_type=jnp.float32) + b
    r1 = jnp.dot(x1_ref[0], w, preferred_element_type=jnp.float32) + b
    o_ref[:, 0, :] = r0
    o_ref[:, 1, :] = r1


def _xproj(x, w_t, b):
    return pl.pallas_call(
        _xproj_body,
        grid=(T // _XP_BLK,),
        in_specs=[
            pl.BlockSpec((1, _XP_BLK, C), lambda i: (0, i, 0)),
            pl.BlockSpec((1, _XP_BLK, C), lambda i: (1, i, 0)),
            pl.BlockSpec((C, C3), lambda i: (0, 0)),
            pl.BlockSpec((1, C3), lambda i: (0, 0)),
        ],
        out_specs=pl.BlockSpec((_XP_BLK, B, C3), lambda i: (i, 0, 0)),
        out_shape=jax.ShapeDtypeStruct((T, B, C3), jnp.float32),
    )(x, x, w_t, b.reshape(1, C3))


_GRU_CHUNK = 128


def _gru_body(xp_ref, whh_ref, bhh_ref, o_ref, h_scr):
    @pl.when(pl.program_id(0) == 0)
    def _init():
        h_scr[...] = jnp.zeros_like(h_scr)

    whh = whh_ref[...]
    bhh = bhh_ref[...]

    def step(t, h):
        xp = xp_ref[pl.ds(t, 1)][0]
        gh = (
            jnp.dot(
                h.astype(jnp.bfloat16), whh,
                preferred_element_type=jnp.float32,
            )
            + bhh
        )
        r = jax.nn.sigmoid(xp[:, :C] + gh[:, :C])
        z = jax.nn.sigmoid(xp[:, C : 2 * C] + gh[:, C : 2 * C])
        n = jnp.tanh(xp[:, 2 * C :] + r * gh[:, 2 * C :])
        h_new = (1.0 - z) * n + z * h
        o_ref[pl.ds(t, 1)] = h_new[None]
        return h_new

    h_fin = jax.lax.fori_loop(0, _GRU_CHUNK, step, h_scr[...], unroll=8)
    h_scr[...] = h_fin


def _gru(xp3, whh_t, bhh):
    return pl.pallas_call(
        _gru_body,
        grid=(T // _GRU_CHUNK,),
        in_specs=[
            pl.BlockSpec((_GRU_CHUNK, B, C3), lambda i: (i, 0, 0)),
            pl.BlockSpec((C, C3), lambda i: (0, 0)),
            pl.BlockSpec((1, C3), lambda i: (0, 0)),
        ],
        out_specs=pl.BlockSpec((_GRU_CHUNK, B, C), lambda i: (i, 0, 0)),
        out_shape=jax.ShapeDtypeStruct((T, B, C), jnp.float32),
        scratch_shapes=[pltpu.VMEM((B, C), jnp.float32)],
    )(xp3, whh_t, bhh.reshape(1, C3))


def _gate_body(h_ref, w1_ref, b1_ref, w2_ref, b2_ref, e_ref):
    g = jnp.tanh(
        jnp.dot(h_ref[...], w1_ref[...], preferred_element_type=jnp.float32)
        + b1_ref[...]
    )
    logit = jnp.sum(g * w2_ref[...], axis=1, keepdims=True) + b2_ref[...]
    e_ref[...] = jax.nn.sigmoid(logit)


def _gate(hflat, w1_t, b1, w2, b2, blk=512):
    m = hflat.shape[0]
    return pl.pallas_call(
        _gate_body,
        grid=(m // blk,),
        in_specs=[
            pl.BlockSpec((blk, C), lambda i: (i, 0)),
            pl.BlockSpec((C, GATE_H), lambda i: (0, 0)),
            pl.BlockSpec((1, GATE_H), lambda i: (0, 0)),
            pl.BlockSpec((1, GATE_H), lambda i: (0, 0)),
            pl.BlockSpec((1, 1), lambda i: (0, 0)),
        ],
        out_specs=pl.BlockSpec((blk, 1), lambda i: (i, 0)),
        out_shape=jax.ShapeDtypeStruct((m, 1), jnp.float32),
    )(hflat, w1_t, b1.reshape(1, GATE_H), w2.reshape(1, GATE_H), b2.reshape(1, 1))


_SEL_BLK = 256


def _sel_body(eTB_ref, eBT_ref, sel_ref, cnt_ref):
    cnt_ref[...] = jnp.sum(
        (eTB_ref[...] > 0.5).astype(jnp.float32), axis=(0, 1), keepdims=True
    )
    for b in range(B):
        e_row = eBT_ref[b : b + 1, :]
        for i in range(T // _SEL_BLK):
            e_col = eTB_ref[pl.ds(i * _SEL_BLK, _SEL_BLK), b : b + 1]
            jj = jax.lax.broadcasted_iota(jnp.int32, (_SEL_BLK, T), 1)
            ii = i * _SEL_BLK + jax.lax.broadcasted_iota(
                jnp.int32, (_SEL_BLK, T), 0
            )
            gt = (e_row > e_col).astype(jnp.float32)
            eq = ((e_row == e_col) & (jj < ii)).astype(jnp.float32)
            rank = jnp.sum(gt + eq, axis=1, keepdims=True)
            sel_ref[pl.ds(i * _SEL_BLK, _SEL_BLK), b : b + 1] = (
                rank < float(K_MAX)
            ).astype(jnp.float32)


def _select(e_TB, e_BT):
    return pl.pallas_call(
        _sel_body,
        grid=(1,),
        in_specs=[
            pl.BlockSpec((T, B), lambda i: (0, 0)),
            pl.BlockSpec((B, T), lambda i: (0, 0)),
        ],
        out_specs=[
            pl.BlockSpec((T, B), lambda i: (0, 0)),
            pl.BlockSpec((1, 1), lambda i: (0, 0)),
        ],
        out_shape=[
            jax.ShapeDtypeStruct((T, B), jnp.float32),
            jax.ShapeDtypeStruct((1, 1), jnp.float32),
        ],
    )(e_TB, e_BT)


_QKV_BLK = 256


def _qkv_body(h_ref, w_ref, b_ref, q_ref, k_ref, v_ref):
    hm = h_ref[...].reshape(_QKV_BLK * B, C)
    r = (
        jnp.dot(hm, w_ref[...], preferred_element_type=jnp.float32)
        + b_ref[...]
    )
    rb = r.astype(jnp.bfloat16)
    q_ref[...] = rb[:, :C].reshape(_QKV_BLK, B, C)
    k_ref[...] = rb[:, C : 2 * C].reshape(_QKV_BLK, B, C)
    v_ref[...] = rb[:, 2 * C :].reshape(_QKV_BLK, B, C)


def _qkv(hr3, w_t, b):
    spec = pl.BlockSpec((_QKV_BLK, B, C), lambda i: (i, 0, 0))
    shp = jax.ShapeDtypeStruct((T, B, C), jnp.bfloat16)
    return pl.pallas_call(
        _qkv_body,
        grid=(T // _QKV_BLK,),
        in_specs=[
            spec,
            pl.BlockSpec((C, C3), lambda i: (0, 0)),
            pl.BlockSpec((1, C3), lambda i: (0, 0)),
        ],
        out_specs=[spec, spec, spec],
        out_shape=[shp, shp, shp],
    )(hr3, w_t, b.reshape(1, C3))


_TQ = 256


def _attn_body(q_ref, k_ref, v_ref, sel_ref, o_ref):
    qi = pl.program_id(0)
    scale = 1.0 / math.sqrt(HD)
    ig = qi * _TQ + jax.lax.broadcasted_iota(jnp.int32, (_TQ, T), 0)
    jg = jax.lax.broadcasted_iota(jnp.int32, (_TQ, T), 1)
    causal = jg <= ig
    diag = jg == ig
    for b in range(B):
        selr = sel_ref[b]
        allowed = ((selr > 0.5) & causal) | diag
        for h in range(H):
            lo = b * C + h * HD
            q = q_ref[:, lo : lo + HD]
            k = k_ref[:, lo : lo + HD]
            v = v_ref[:, lo : lo + HD]
            s = (
                jax.lax.dot_general(
                    q, k, (((1,), (1,)), ((), ())),
                    preferred_element_type=jnp.float32,
                )
                * scale
            )
            s = jnp.where(allowed, s, NEG)
            m = jnp.max(s, axis=1, keepdims=True)
            p = jnp.exp(s - m)
            l = jnp.sum(p, axis=1, keepdims=True)
            o_ref[:, lo : lo + HD] = jnp.dot(
                p.astype(jnp.bfloat16), v,
                preferred_element_type=jnp.float32,
            ) / l


def _attn(q2, k2, v2, sel3):
    return pl.pallas_call(
        _attn_body,
        grid=(T // _TQ,),
        in_specs=[
            pl.BlockSpec((_TQ, B * C), lambda i: (i, 0)),
            pl.BlockSpec((T, B * C), lambda i: (0, 0)),
            pl.BlockSpec((T, B * C), lambda i: (0, 0)),
            pl.BlockSpec((B, 1, T), lambda i: (0, 0, 0)),
        ],
        out_specs=pl.BlockSpec((_TQ, B * C), lambda i: (i, 0)),
        out_shape=jax.ShapeDtypeStruct((T, B * C), jnp.float32),
    )(q2, k2, v2, sel3)


_OUT_BLK = 256


def _out_body(y_ref, pw_ref, pb_ref, e_ref, s_ref, h_ref,
              lw_ref, lb_ref, w1_ref, b1_ref, w2_ref, b2_ref, o_ref):
    ym = y_ref[...].reshape(_OUT_BLK * B, C)
    p = (
        jnp.dot(ym, pw_ref[...], preferred_element_type=jnp.float32)
        + pb_ref[...]
    ).reshape(_OUT_BLK, B, C)
    hm = h_ref[...] + p * e_ref[...] * s_ref[...]
    mu = jnp.mean(hm, axis=2, keepdims=True)
    d = hm - mu
    var = jnp.mean(d * d, axis=2, keepdims=True)
    hn = d / jnp.sqrt(var + 1e-5) * lw_ref[...] + lb_ref[...]
    hnm = hn.reshape(_OUT_BLK * B, C)
    f = (
        jnp.dot(hnm, w1_ref[...], preferred_element_type=jnp.float32)
        + b1_ref[...]
    )
    g = 0.5 * f * (1.0 + jax.lax.erf(f * (1.0 / math.sqrt(2.0))))
    ffn = (
        jnp.dot(g, w2_ref[...], preferred_element_type=jnp.float32)
        + b2_ref[...]
    )
    o_ref[...] = hn + ffn.reshape(_OUT_BLK, B, C)


def _out_fused(y3, proj_w_t, proj_b, e3, s3, h3, ln_w, ln_b,
               w1_t, b1, w2_t, b2):
    blkspec = pl.BlockSpec((_OUT_BLK, B, C), lambda i: (i, 0, 0))
    return pl.pallas_call(
        _out_body,
        grid=(T // _OUT_BLK,),
        in_specs=[
            blkspec,
            pl.BlockSpec((C, C), lambda i: (0, 0)),
            pl.BlockSpec((1, C), lambda i: (0, 0)),
            pl.BlockSpec((_OUT_BLK, B, 1), lambda i: (i, 0, 0)),
            pl.BlockSpec((_OUT_BLK, B, 1), lambda i: (i, 0, 0)),
            blkspec,
            pl.BlockSpec((1, 1, C), lambda i: (0, 0, 0)),
            pl.BlockSpec((1, 1, C), lambda i: (0, 0, 0)),
            pl.BlockSpec((C, F), lambda i: (0, 0)),
            pl.BlockSpec((1, F), lambda i: (0, 0)),
            pl.BlockSpec((F, C), lambda i: (0, 0)),
            pl.BlockSpec((1, C), lambda i: (0, 0)),
        ],
        out_specs=blkspec,
        out_shape=jax.ShapeDtypeStruct((T, B, C), jnp.float32),
    )(y3, proj_w_t, proj_b.reshape(1, C), e3, s3, h3,
      ln_w.reshape(1, 1, C), ln_b.reshape(1, 1, C),
      w1_t, b1.reshape(1, F), w2_t, b2.reshape(1, C))


@jax.jit
def kernel(x, W_ih, W_hh, b_ih, b_hh, gate_W1, gate_b1, gate_W2, gate_b2,
           qkv_W, qkv_b, proj_W, proj_b, ln_w, ln_b,
           ffn_W1, ffn_b1, ffn_W2, ffn_b2):
    xp3 = _xproj(x, W_ih.T, b_ih)
    hr3 = _gru(xp3, W_hh.T.astype(jnp.bfloat16), b_hh)
    hflat = hr3.reshape(T * B, C)

    e_flat = _gate(hflat, gate_W1.T, gate_b1, gate_W2, gate_b2)
    e_TB = e_flat.reshape(T, B)
    sel_TB, cnt = _select(e_TB, e_TB.T)

    q2, k2, v2 = _qkv(hr3, qkv_W.T, qkv_b)
    y2 = _attn(q2.reshape(T, B * C), k2.reshape(T, B * C),
               v2.reshape(T, B * C), sel_TB.T.reshape(B, 1, T))

    hout = _out_fused(y2.reshape(T, B, C), proj_W.T, proj_b,
                      e_flat.reshape(T, B, 1), sel_TB.reshape(T, B, 1),
                      hr3, ln_w, ln_b, ffn_W1.T, ffn_b1, ffn_W2.T, ffn_b2)

    h = hout.transpose(1, 0, 2)
    energy = e_TB.T.reshape(B, T, 1)
    return (h, energy, cnt[0, 0])


# speedup vs baseline: 1.0195x; 1.0195x over previous
import math

import jax
import jax.numpy as jnp
from jax.experimental import pallas as pl
from jax.experimental.pallas import tpu as pltpu

B = 2
T = 2048
C = 768
C3 = 3 * C
H = 12
HD = C // H
K_MAX = T // 2
F = 4 * C
GATE_H = 32

NEG = -1e30


_XP_BLK = 256


def _xproj_body(x0_ref, x1_ref, w_ref, b_ref, o_ref):
    w = w_ref[...]
    b = b_ref[...]
    r0 = jnp.dot(x0_ref[0], w, preferred_element_type=jnp.float32) + b
    r1 = jnp.dot(x1_ref[0], w, preferred_element_type=jnp.float32) + b
    o_ref[:, 0, :] = r0
    o_ref[:, 1, :] = r1


def _xproj(x, w_t, b):
    return pl.pallas_call(
        _xproj_body,
        grid=(T // _XP_BLK,),
        in_specs=[
            pl.BlockSpec((1, _XP_BLK, C), lambda i: (0, i, 0)),
            pl.BlockSpec((1, _XP_BLK, C), lambda i: (1, i, 0)),
            pl.BlockSpec((C, C3), lambda i: (0, 0)),
            pl.BlockSpec((1, C3), lambda i: (0, 0)),
        ],
        out_specs=pl.BlockSpec((_XP_BLK, B, C3), lambda i: (i, 0, 0)),
        out_shape=jax.ShapeDtypeStruct((T, B, C3), jnp.float32),
    )(x, x, w_t, b.reshape(1, C3))


_GRU_CHUNK = 128


def _gru_body(xp_ref, whh_ref, bhh_ref, o_ref, h_scr):
    @pl.when(pl.program_id(0) == 0)
    def _init():
        h_scr[...] = jnp.zeros_like(h_scr)

    whh = whh_ref[...]
    bhh = bhh_ref[...]
    _U = 8

    def step8(i, h):
        xp8 = xp_ref[pl.ds(i * _U, _U)]
        outs = []
        for j in range(_U):
            xp = xp8[j]
            gh = (
                jnp.dot(
                    h.astype(jnp.bfloat16), whh,
                    preferred_element_type=jnp.float32,
                )
                + bhh
            )
            r = jax.nn.sigmoid(xp[:, :C] + gh[:, :C])
            z = jax.nn.sigmoid(xp[:, C : 2 * C] + gh[:, C : 2 * C])
            n = jnp.tanh(xp[:, 2 * C :] + r * gh[:, 2 * C :])
            h = (1.0 - z) * n + z * h
            outs.append(h[None])
        o_ref[pl.ds(i * _U, _U)] = jnp.concatenate(outs, axis=0)
        return h

    h_fin = jax.lax.fori_loop(0, _GRU_CHUNK // _U, step8, h_scr[...])
    h_scr[...] = h_fin


def _gru(xp3, whh_t, bhh):
    return pl.pallas_call(
        _gru_body,
        grid=(T // _GRU_CHUNK,),
        in_specs=[
            pl.BlockSpec((_GRU_CHUNK, B, C3), lambda i: (i, 0, 0)),
            pl.BlockSpec((C, C3), lambda i: (0, 0)),
            pl.BlockSpec((1, C3), lambda i: (0, 0)),
        ],
        out_specs=pl.BlockSpec((_GRU_CHUNK, B, C), lambda i: (i, 0, 0)),
        out_shape=jax.ShapeDtypeStruct((T, B, C), jnp.float32),
        scratch_shapes=[pltpu.VMEM((B, C), jnp.float32)],
    )(xp3, whh_t, bhh.reshape(1, C3))


def _gate_body(h_ref, w1_ref, b1_ref, w2_ref, b2_ref, e_ref):
    g = jnp.tanh(
        jnp.dot(h_ref[...], w1_ref[...], preferred_element_type=jnp.float32)
        + b1_ref[...]
    )
    logit = jnp.sum(g * w2_ref[...], axis=1, keepdims=True) + b2_ref[...]
    e_ref[...] = jax.nn.sigmoid(logit)


def _gate(hflat, w1_t, b1, w2, b2, blk=512):
    m = hflat.shape[0]
    return pl.pallas_call(
        _gate_body,
        grid=(m // blk,),
        in_specs=[
            pl.BlockSpec((blk, C), lambda i: (i, 0)),
            pl.BlockSpec((C, GATE_H), lambda i: (0, 0)),
            pl.BlockSpec((1, GATE_H), lambda i: (0, 0)),
            pl.BlockSpec((1, GATE_H), lambda i: (0, 0)),
            pl.BlockSpec((1, 1), lambda i: (0, 0)),
        ],
        out_specs=pl.BlockSpec((blk, 1), lambda i: (i, 0)),
        out_shape=jax.ShapeDtypeStruct((m, 1), jnp.float32),
    )(hflat, w1_t, b1.reshape(1, GATE_H), w2.reshape(1, GATE_H), b2.reshape(1, 1))


_SEL_BLK = 256


def _sel_body(eTB_ref, eBT_ref, sel_ref, cnt_ref):
    cnt_ref[...] = jnp.sum(
        (eTB_ref[...] > 0.5).astype(jnp.float32), axis=(0, 1), keepdims=True
    )
    for b in range(B):
        e_row = eBT_ref[b : b + 1, :]
        for i in range(T // _SEL_BLK):
            e_col = eTB_ref[pl.ds(i * _SEL_BLK, _SEL_BLK), b : b + 1]
            jj = jax.lax.broadcasted_iota(jnp.int32, (_SEL_BLK, T), 1)
            ii = i * _SEL_BLK + jax.lax.broadcasted_iota(
                jnp.int32, (_SEL_BLK, T), 0
            )
            gt = (e_row > e_col).astype(jnp.float32)
            eq = ((e_row == e_col) & (jj < ii)).astype(jnp.float32)
            rank = jnp.sum(gt + eq, axis=1, keepdims=True)
            sel_ref[pl.ds(i * _SEL_BLK, _SEL_BLK), b : b + 1] = (
                rank < float(K_MAX)
            ).astype(jnp.float32)


def _select(e_TB, e_BT):
    return pl.pallas_call(
        _sel_body,
        grid=(1,),
        in_specs=[
            pl.BlockSpec((T, B), lambda i: (0, 0)),
            pl.BlockSpec((B, T), lambda i: (0, 0)),
        ],
        out_specs=[
            pl.BlockSpec((T, B), lambda i: (0, 0)),
            pl.BlockSpec((1, 1), lambda i: (0, 0)),
        ],
        out_shape=[
            jax.ShapeDtypeStruct((T, B), jnp.float32),
            jax.ShapeDtypeStruct((1, 1), jnp.float32),
        ],
    )(e_TB, e_BT)


_QKV_BLK = 256


def _qkv_body(h_ref, w_ref, b_ref, q_ref, k_ref, v_ref):
    hm = h_ref[...].reshape(_QKV_BLK * B, C)
    r = (
        jnp.dot(hm, w_ref[...], preferred_element_type=jnp.float32)
        + b_ref[...]
    )
    rb = r.astype(jnp.bfloat16)
    q_ref[...] = rb[:, :C].reshape(_QKV_BLK, B, C)
    k_ref[...] = rb[:, C : 2 * C].reshape(_QKV_BLK, B, C)
    v_ref[...] = rb[:, 2 * C :].reshape(_QKV_BLK, B, C)


def _qkv(hr3, w_t, b):
    spec = pl.BlockSpec((_QKV_BLK, B, C), lambda i: (i, 0, 0))
    shp = jax.ShapeDtypeStruct((T, B, C), jnp.bfloat16)
    return pl.pallas_call(
        _qkv_body,
        grid=(T // _QKV_BLK,),
        in_specs=[
            spec,
            pl.BlockSpec((C, C3), lambda i: (0, 0)),
            pl.BlockSpec((1, C3), lambda i: (0, 0)),
        ],
        out_specs=[spec, spec, spec],
        out_shape=[shp, shp, shp],
    )(hr3, w_t, b.reshape(1, C3))


_TQ = 256


def _attn_body(q_ref, k_ref, v_ref, sel_ref, o_ref):
    qi = pl.program_id(0)
    scale = 1.0 / math.sqrt(HD)
    ig = qi * _TQ + jax.lax.broadcasted_iota(jnp.int32, (_TQ, T), 0)
    jg = jax.lax.broadcasted_iota(jnp.int32, (_TQ, T), 1)
    causal = jg <= ig
    diag = jg == ig
    for b in range(B):
        selr = sel_ref[b]
        allowed = ((selr > 0.5) & causal) | diag
        for h in range(H):
            lo = b * C + h * HD
            q = q_ref[:, lo : lo + HD]
            k = k_ref[:, lo : lo + HD]
            v = v_ref[:, lo : lo + HD]
            s = (
                jax.lax.dot_general(
                    q, k, (((1,), (1,)), ((), ())),
                    preferred_element_type=jnp.float32,
                )
                * scale
            )
            s = jnp.where(allowed, s, NEG)
            m = jnp.max(s, axis=1, keepdims=True)
            p = jnp.exp(s - m)
            l = jnp.sum(p, axis=1, keepdims=True)
            o_ref[:, lo : lo + HD] = jnp.dot(
                p, v.astype(jnp.float32),
                preferred_element_type=jnp.float32,
            ) / l


def _attn(q2, k2, v2, sel3):
    return pl.pallas_call(
        _attn_body,
        grid=(T // _TQ,),
        in_specs=[
            pl.BlockSpec((_TQ, B * C), lambda i: (i, 0)),
            pl.BlockSpec((T, B * C), lambda i: (0, 0)),
            pl.BlockSpec((T, B * C), lambda i: (0, 0)),
            pl.BlockSpec((B, 1, T), lambda i: (0, 0, 0)),
        ],
        out_specs=pl.BlockSpec((_TQ, B * C), lambda i: (i, 0)),
        out_shape=jax.ShapeDtypeStruct((T, B * C), jnp.float32),
    )(q2, k2, v2, sel3)


_OUT_BLK = 256


def _out_body(y_ref, pw_ref, pb_ref, e_ref, s_ref, h_ref,
              lw_ref, lb_ref, w1_ref, b1_ref, w2_ref, b2_ref, o_ref):
    ym = y_ref[...].reshape(_OUT_BLK * B, C)
    p = (
        jnp.dot(ym, pw_ref[...], preferred_element_type=jnp.float32)
        + pb_ref[...]
    ).reshape(_OUT_BLK, B, C)
    hm = h_ref[...] + p * e_ref[...] * s_ref[...]
    mu = jnp.mean(hm, axis=2, keepdims=True)
    d = hm - mu
    var = jnp.mean(d * d, axis=2, keepdims=True)
    hn = d / jnp.sqrt(var + 1e-5) * lw_ref[...] + lb_ref[...]
    hnm = hn.reshape(_OUT_BLK * B, C)
    f = (
        jnp.dot(hnm, w1_ref[...], preferred_element_type=jnp.float32)
        + b1_ref[...]
    )
    g = 0.5 * f * (1.0 + jax.lax.erf(f * (1.0 / math.sqrt(2.0))))
    ffn = (
        jnp.dot(g, w2_ref[...], preferred_element_type=jnp.float32)
        + b2_ref[...]
    )
    o_ref[...] = hn + ffn.reshape(_OUT_BLK, B, C)


def _out_fused(y3, proj_w_t, proj_b, e3, s3, h3, ln_w, ln_b,
               w1_t, b1, w2_t, b2):
    blkspec = pl.BlockSpec((_OUT_BLK, B, C), lambda i: (i, 0, 0))
    return pl.pallas_call(
        _out_body,
        grid=(T // _OUT_BLK,),
        in_specs=[
            blkspec,
            pl.BlockSpec((C, C), lambda i: (0, 0)),
            pl.BlockSpec((1, C), lambda i: (0, 0)),
            pl.BlockSpec((_OUT_BLK, B, 1), lambda i: (i, 0, 0)),
            pl.BlockSpec((_OUT_BLK, B, 1), lambda i: (i, 0, 0)),
            blkspec,
            pl.BlockSpec((1, 1, C), lambda i: (0, 0, 0)),
            pl.BlockSpec((1, 1, C), lambda i: (0, 0, 0)),
            pl.BlockSpec((C, F), lambda i: (0, 0)),
            pl.BlockSpec((1, F), lambda i: (0, 0)),
            pl.BlockSpec((F, C), lambda i: (0, 0)),
            pl.BlockSpec((1, C), lambda i: (0, 0)),
        ],
        out_specs=blkspec,
        out_shape=jax.ShapeDtypeStruct((T, B, C), jnp.float32),
    )(y3, proj_w_t, proj_b.reshape(1, C), e3, s3, h3,
      ln_w.reshape(1, 1, C), ln_b.reshape(1, 1, C),
      w1_t, b1.reshape(1, F), w2_t, b2.reshape(1, C))


@jax.jit
def kernel(x, W_ih, W_hh, b_ih, b_hh, gate_W1, gate_b1, gate_W2, gate_b2,
           qkv_W, qkv_b, proj_W, proj_b, ln_w, ln_b,
           ffn_W1, ffn_b1, ffn_W2, ffn_b2):
    xp3 = _xproj(x, W_ih.T, b_ih)
    hr3 = _gru(xp3, W_hh.T.astype(jnp.bfloat16), b_hh)
    hflat = hr3.reshape(T * B, C)

    e_flat = _gate(hflat, gate_W1.T, gate_b1, gate_W2, gate_b2)
    e_TB = e_flat.reshape(T, B)
    sel_TB, cnt = _select(e_TB, e_TB.T)

    q2, k2, v2 = _qkv(hr3, qkv_W.T, qkv_b)
    y2 = _attn(q2.reshape(T, B * C), k2.reshape(T, B * C),
               v2.reshape(T, B * C), sel_TB.T.reshape(B, 1, T))

    hout = _out_fused(y2.reshape(T, B, C), proj_W.T, proj_b,
                      e_flat.reshape(T, B, 1), sel_TB.reshape(T, B, 1),
                      hr3, ln_w, ln_b, ffn_W1.T, ffn_b1, ffn_W2.T, ffn_b2)

    h = hout.transpose(1, 0, 2)
    energy = e_TB.T.reshape(B, T, 1)
    return (h, energy, cnt[0, 0])


# speedup vs baseline: 1.0486x; 1.0286x over previous
import math

import jax
import jax.numpy as jnp
from jax.experimental import pallas as pl
from jax.experimental.pallas import tpu as pltpu

B = 2
T = 2048
C = 768
C3 = 3 * C
H = 12
HD = C // H
K_MAX = T // 2
F = 4 * C
GATE_H = 32

NEG = -1e30


_XP_BLK = 256


def _xproj_body(x0_ref, x1_ref, w_ref, b_ref, o_ref):
    w = w_ref[...]
    b = b_ref[...]
    r0 = jnp.dot(x0_ref[0], w, preferred_element_type=jnp.float32) + b
    r1 = jnp.dot(x1_ref[0], w, preferred_element_type=jnp.float32) + b
    o_ref[:, 0, :] = r0
    o_ref[:, 1, :] = r1


def _xproj(x, w_t, b):
    return pl.pallas_call(
        _xproj_body,
        grid=(T // _XP_BLK,),
        in_specs=[
            pl.BlockSpec((1, _XP_BLK, C), lambda i: (0, i, 0)),
            pl.BlockSpec((1, _XP_BLK, C), lambda i: (1, i, 0)),
            pl.BlockSpec((C, C3), lambda i: (0, 0)),
            pl.BlockSpec((1, C3), lambda i: (0, 0)),
        ],
        out_specs=pl.BlockSpec((_XP_BLK, B, C3), lambda i: (i, 0, 0)),
        out_shape=jax.ShapeDtypeStruct((T, B, C3), jnp.float32),
    )(x, x, w_t, b.reshape(1, C3))


_GRU_CHUNK = 128


def _gru_body(xp_ref, whh_ref, bhh_ref, o_ref, h_scr):
    @pl.when(pl.program_id(0) == 0)
    def _init():
        h_scr[...] = jnp.zeros_like(h_scr)

    whh = whh_ref[...]
    bhh = bhh_ref[...]
    _U = 8

    def step8(i, h):
        xp8 = xp_ref[pl.ds(i * _U, _U)]
        outs = []
        for j in range(_U):
            xp = xp8[j]
            gh = (
                jnp.dot(
                    h.astype(jnp.bfloat16), whh,
                    preferred_element_type=jnp.float32,
                )
                + bhh
            )
            r = jax.nn.sigmoid(xp[:, :C] + gh[:, :C])
            z = jax.nn.sigmoid(xp[:, C : 2 * C] + gh[:, C : 2 * C])
            n = jnp.tanh(xp[:, 2 * C :] + r * gh[:, 2 * C :])
            h = (1.0 - z) * n + z * h
            outs.append(h[None])
        o_ref[pl.ds(i * _U, _U)] = jnp.concatenate(outs, axis=0)
        return h

    h_fin = jax.lax.fori_loop(0, _GRU_CHUNK // _U, step8, h_scr[...])
    h_scr[...] = h_fin


def _gru(xp3, whh_t, bhh):
    return pl.pallas_call(
        _gru_body,
        grid=(T // _GRU_CHUNK,),
        in_specs=[
            pl.BlockSpec((_GRU_CHUNK, B, C3), lambda i: (i, 0, 0)),
            pl.BlockSpec((C, C3), lambda i: (0, 0)),
            pl.BlockSpec((1, C3), lambda i: (0, 0)),
        ],
        out_specs=pl.BlockSpec((_GRU_CHUNK, B, C), lambda i: (i, 0, 0)),
        out_shape=jax.ShapeDtypeStruct((T, B, C), jnp.float32),
        scratch_shapes=[pltpu.VMEM((B, C), jnp.float32)],
    )(xp3, whh_t, bhh.reshape(1, C3))


def _gate_body(h_ref, w1_ref, b1_ref, w2_ref, b2_ref, e_ref):
    g = jnp.tanh(
        jnp.dot(h_ref[...], w1_ref[...], preferred_element_type=jnp.float32)
        + b1_ref[...]
    )
    logit = jnp.sum(g * w2_ref[...], axis=1, keepdims=True) + b2_ref[...]
    e_ref[...] = jax.nn.sigmoid(logit)


def _gate(hflat, w1_t, b1, w2, b2, blk=512):
    m = hflat.shape[0]
    return pl.pallas_call(
        _gate_body,
        grid=(m // blk,),
        in_specs=[
            pl.BlockSpec((blk, C), lambda i: (i, 0)),
            pl.BlockSpec((C, GATE_H), lambda i: (0, 0)),
            pl.BlockSpec((1, GATE_H), lambda i: (0, 0)),
            pl.BlockSpec((1, GATE_H), lambda i: (0, 0)),
            pl.BlockSpec((1, 1), lambda i: (0, 0)),
        ],
        out_specs=pl.BlockSpec((blk, 1), lambda i: (i, 0)),
        out_shape=jax.ShapeDtypeStruct((m, 1), jnp.float32),
    )(hflat, w1_t, b1.reshape(1, GATE_H), w2.reshape(1, GATE_H), b2.reshape(1, 1))


_SEL_BLK = 256


def _sel_body(eTB_ref, eBT_ref, sel_ref, cnt_ref):
    cnt_ref[...] = jnp.sum(
        (eTB_ref[...] > 0.5).astype(jnp.float32), axis=(0, 1), keepdims=True
    )
    for b in range(B):
        e_row = eBT_ref[b : b + 1, :]
        for i in range(T // _SEL_BLK):
            e_col = eTB_ref[pl.ds(i * _SEL_BLK, _SEL_BLK), b : b + 1]
            jj = jax.lax.broadcasted_iota(jnp.int32, (_SEL_BLK, T), 1)
            ii = i * _SEL_BLK + jax.lax.broadcasted_iota(
                jnp.int32, (_SEL_BLK, T), 0
            )
            gt = (e_row > e_col).astype(jnp.float32)
            eq = ((e_row == e_col) & (jj < ii)).astype(jnp.float32)
            rank = jnp.sum(gt + eq, axis=1, keepdims=True)
            sel_ref[pl.ds(i * _SEL_BLK, _SEL_BLK), b : b + 1] = (
                rank < float(K_MAX)
            ).astype(jnp.float32)


def _select(e_TB, e_BT):
    return pl.pallas_call(
        _sel_body,
        grid=(1,),
        in_specs=[
            pl.BlockSpec((T, B), lambda i: (0, 0)),
            pl.BlockSpec((B, T), lambda i: (0, 0)),
        ],
        out_specs=[
            pl.BlockSpec((T, B), lambda i: (0, 0)),
            pl.BlockSpec((1, 1), lambda i: (0, 0)),
        ],
        out_shape=[
            jax.ShapeDtypeStruct((T, B), jnp.float32),
            jax.ShapeDtypeStruct((1, 1), jnp.float32),
        ],
    )(e_TB, e_BT)


_QKV_BLK = 256


def _qkv_body(h_ref, w_ref, b_ref, gw1_ref, gb1_ref, gw2_ref, gb2_ref,
              q_ref, k_ref, v_ref, e_ref):
    hm = h_ref[...].reshape(_QKV_BLK * B, C)
    r = (
        jnp.dot(hm, w_ref[...], preferred_element_type=jnp.float32)
        + b_ref[...]
    )
    rb = r.astype(jnp.bfloat16)
    q_ref[...] = rb[:, :C].reshape(_QKV_BLK, B, C)
    k_ref[...] = rb[:, C : 2 * C].reshape(_QKV_BLK, B, C)
    v_ref[...] = rb[:, 2 * C :].reshape(_QKV_BLK, B, C)
    g = jnp.tanh(
        jnp.dot(hm, gw1_ref[...], preferred_element_type=jnp.float32)
        + gb1_ref[...]
    )
    logit = jnp.sum(g * gw2_ref[...], axis=1, keepdims=True) + gb2_ref[...]
    e_ref[...] = jax.nn.sigmoid(logit).reshape(_QKV_BLK, B, 1)


def _qkv(hr3, w_t, b, gw1_t, gb1, gw2, gb2):
    spec = pl.BlockSpec((_QKV_BLK, B, C), lambda i: (i, 0, 0))
    shp = jax.ShapeDtypeStruct((T, B, C), jnp.bfloat16)
    return pl.pallas_call(
        _qkv_body,
        grid=(T // _QKV_BLK,),
        in_specs=[
            spec,
            pl.BlockSpec((C, C3), lambda i: (0, 0)),
            pl.BlockSpec((1, C3), lambda i: (0, 0)),
            pl.BlockSpec((C, GATE_H), lambda i: (0, 0)),
            pl.BlockSpec((1, GATE_H), lambda i: (0, 0)),
            pl.BlockSpec((1, GATE_H), lambda i: (0, 0)),
            pl.BlockSpec((1, 1), lambda i: (0, 0)),
        ],
        out_specs=[spec, spec, spec,
                   pl.BlockSpec((_QKV_BLK, B, 1), lambda i: (i, 0, 0))],
        out_shape=[shp, shp, shp,
                   jax.ShapeDtypeStruct((T, B, 1), jnp.float32)],
    )(hr3, w_t, b.reshape(1, C3), gw1_t, gb1.reshape(1, GATE_H),
      gw2.reshape(1, GATE_H), gb2.reshape(1, 1))


_TQ = 256


def _attn_body(q_ref, k_ref, v_ref, sel_ref, o_ref):
    qi = pl.program_id(0)
    scale = 1.0 / math.sqrt(HD)
    ig = qi * _TQ + jax.lax.broadcasted_iota(jnp.int32, (_TQ, T), 0)
    jg = jax.lax.broadcasted_iota(jnp.int32, (_TQ, T), 1)
    causal = jg <= ig
    diag = jg == ig
    for b in range(B):
        selr = sel_ref[b]
        allowed = ((selr > 0.5) & causal) | diag
        for h in range(H):
            lo = b * C + h * HD
            q = q_ref[:, lo : lo + HD]
            k = k_ref[:, lo : lo + HD]
            v = v_ref[:, lo : lo + HD]
            s = (
                jax.lax.dot_general(
                    q, k, (((1,), (1,)), ((), ())),
                    preferred_element_type=jnp.float32,
                )
                * scale
            )
            s = jnp.where(allowed, s, NEG)
            m = jnp.max(s, axis=1, keepdims=True)
            p = jnp.exp(s - m)
            l = jnp.sum(p, axis=1, keepdims=True)
            o_ref[:, lo : lo + HD] = jnp.dot(
                p, v.astype(jnp.float32),
                preferred_element_type=jnp.float32,
            ) / l


def _attn(q2, k2, v2, sel3):
    return pl.pallas_call(
        _attn_body,
        grid=(T // _TQ,),
        in_specs=[
            pl.BlockSpec((_TQ, B * C), lambda i: (i, 0)),
            pl.BlockSpec((T, B * C), lambda i: (0, 0)),
            pl.BlockSpec((T, B * C), lambda i: (0, 0)),
            pl.BlockSpec((B, 1, T), lambda i: (0, 0, 0)),
        ],
        out_specs=pl.BlockSpec((_TQ, B * C), lambda i: (i, 0)),
        out_shape=jax.ShapeDtypeStruct((T, B * C), jnp.float32),
    )(q2, k2, v2, sel3)


_OUT_BLK = 256


def _out_body(y_ref, pw_ref, pb_ref, e_ref, s_ref, h_ref,
              lw_ref, lb_ref, w1_ref, b1_ref, w2_ref, b2_ref, o_ref):
    ym = y_ref[...].reshape(_OUT_BLK * B, C)
    p = (
        jnp.dot(ym, pw_ref[...], preferred_element_type=jnp.float32)
        + pb_ref[...]
    ).reshape(_OUT_BLK, B, C)
    hm = h_ref[...] + p * e_ref[...] * s_ref[...]
    mu = jnp.mean(hm, axis=2, keepdims=True)
    d = hm - mu
    var = jnp.mean(d * d, axis=2, keepdims=True)
    hn = d / jnp.sqrt(var + 1e-5) * lw_ref[...] + lb_ref[...]
    hnm = hn.reshape(_OUT_BLK * B, C)
    f = (
        jnp.dot(hnm, w1_ref[...], preferred_element_type=jnp.float32)
        + b1_ref[...]
    )
    g = 0.5 * f * (1.0 + jax.lax.erf(f * (1.0 / math.sqrt(2.0))))
    ffn = (
        jnp.dot(g, w2_ref[...], preferred_element_type=jnp.float32)
        + b2_ref[...]
    )
    o_ref[...] = hn + ffn.reshape(_OUT_BLK, B, C)


def _out_fused(y3, proj_w_t, proj_b, e3, s3, h3, ln_w, ln_b,
               w1_t, b1, w2_t, b2):
    blkspec = pl.BlockSpec((_OUT_BLK, B, C), lambda i: (i, 0, 0))
    return pl.pallas_call(
        _out_body,
        grid=(T // _OUT_BLK,),
        in_specs=[
            blkspec,
            pl.BlockSpec((C, C), lambda i: (0, 0)),
            pl.BlockSpec((1, C), lambda i: (0, 0)),
            pl.BlockSpec((_OUT_BLK, B, 1), lambda i: (i, 0, 0)),
            pl.BlockSpec((_OUT_BLK, B, 1), lambda i: (i, 0, 0)),
            blkspec,
            pl.BlockSpec((1, 1, C), lambda i: (0, 0, 0)),
            pl.BlockSpec((1, 1, C), lambda i: (0, 0, 0)),
            pl.BlockSpec((C, F), lambda i: (0, 0)),
            pl.BlockSpec((1, F), lambda i: (0, 0)),
            pl.BlockSpec((F, C), lambda i: (0, 0)),
            pl.BlockSpec((1, C), lambda i: (0, 0)),
        ],
        out_specs=blkspec,
        out_shape=jax.ShapeDtypeStruct((T, B, C), jnp.float32),
    )(y3, proj_w_t, proj_b.reshape(1, C), e3, s3, h3,
      ln_w.reshape(1, 1, C), ln_b.reshape(1, 1, C),
      w1_t, b1.reshape(1, F), w2_t, b2.reshape(1, C))


@jax.jit
def kernel(x, W_ih, W_hh, b_ih, b_hh, gate_W1, gate_b1, gate_W2, gate_b2,
           qkv_W, qkv_b, proj_W, proj_b, ln_w, ln_b,
           ffn_W1, ffn_b1, ffn_W2, ffn_b2):
    xp3 = _xproj(x, W_ih.T, b_ih)
    hr3 = _gru(xp3, W_hh.T.astype(jnp.bfloat16), b_hh)
    q2, k2, v2, e3 = _qkv(hr3, qkv_W.T, qkv_b,
                          gate_W1.T, gate_b1, gate_W2, gate_b2)
    e_TB = e3.reshape(T, B)
    sel_TB, cnt = _select(e_TB, e_TB.T)

    y2 = _attn(q2.reshape(T, B * C), k2.reshape(T, B * C),
               v2.reshape(T, B * C), sel_TB.T.reshape(B, 1, T))

    hout = _out_fused(y2.reshape(T, B, C), proj_W.T, proj_b,
                      e3, sel_TB.reshape(T, B, 1),
                      hr3, ln_w, ln_b, ffn_W1.T, ffn_b1, ffn_W2.T, ffn_b2)

    h = hout.transpose(1, 0, 2)
    energy = e_TB.T.reshape(B, T, 1)
    return (h, energy, cnt[0, 0])
